# baseline (speedup 1.0000x reference)
import numpy as np
import jax
import jax.numpy as jnp
from jax.experimental import pallas as pl
from jax.experimental.pallas import tpu as pltpu

NV = 100
NVP = 104
ND = 2
MSG = 64
SB = 8
NSB = NVP // SB
BB = 8
NB = 64 // BB


def _prep_body(x_ref, pf_ref, ce_ref, ws_ref, wr_ref, wres_ref,
               battr_ref, bres_ref, as_ref, ar_ref, res_ref):
    b, n = x_ref.shape[0], x_ref.shape[1]
    x = x_ref[...].reshape(b * n, 4)
    pf = pf_ref[...].reshape(b * n, 2)
    ce = ce_ref[...].reshape(b * n, 16)
    px, py = x[:, 0:1], x[:, 1:2]
    vx, vy = x[:, 2:3], x[:, 3:4]
    fx, fy = pf[:, 0:1], pf[:, 1:2]
    vn2 = vx * vx + vy * vy
    r = jnp.sqrt(vn2)
    safe = r > 0.0
    rinvsafe = 1.0 / jnp.where(safe, r, 1.0)
    c = jnp.where(safe, vx * rinvsafe, 1.0)
    s = jnp.where(safe, vy * rinvsafe, 0.0)
    lp0 = c * px + s * py
    lp1 = -s * px + c * py
    lv0 = c * vx + s * vy
    lv1 = -s * vx + c * vy
    lf0 = c * fx + s * fy
    lf1 = -s * fx + c * fy
    nvel = jnp.sqrt(vn2 + 1e-8)
    npos = jnp.sqrt(px * px + py * py + 1e-8)
    nfld = jnp.sqrt(fx * fx + fy * fy + 1e-8)
    f31 = jnp.concatenate(
        [lp0, lp1, lv0, lv1, lf0, lf1,
         nvel, npos, c, s, lp0, lp1, lf0, lf1, nfld, ce], axis=1)
    a_s = jnp.dot(f31, ws_ref[...], preferred_element_type=jnp.float32)
    a_r = (jnp.dot(f31, wr_ref[...], preferred_element_type=jnp.float32)
           + battr_ref[...])
    rx = (jnp.dot(f31, wres_ref[...], preferred_element_type=jnp.float32)
          + bres_ref[...])
    as_ref[...] = a_s.reshape(b, n, MSG)
    ar_ref[...] = a_r.reshape(b, n, MSG)
    res_ref[...] = rx.reshape(b, n, MSG)


def _edge_body(as_ref, ar_ref, ps_ref, prt_ref, g_ref,
               w1_ref, b1_ref, w2_ref, b2_ref, wo_ref, bo_ref, o_ref):
    sblk = pl.program_id(1)
    a_s = as_ref[0]
    a_r = ar_ref[0]
    h3 = jax.nn.relu(a_s[:, None, :] + a_r[None, :, :])
    psx, psy = ps_ref[0, :, 0:1], ps_ref[0, :, 1:2]
    prx, pry = prt_ref[0, 0:1, :], prt_ref[0, 1:2, :]
    dx = psx - prx
    dy = psy - pry
    d2 = dx * dx + dy * dy
    dist = jnp.sqrt(d2 + 1e-8)
    rr = jnp.sqrt(d2)
    safe = rr > 0.0
    inv = jnp.where(safe, 1.0 / jnp.where(safe, rr, 1.0), 0.0)
    cphi = jnp.where(safe, dx * inv, 1.0)
    sphi = dy * inv
    w1 = w1_ref[...]
    b1 = b1_ref[...]
    f1 = jax.nn.relu(dist[..., None] * w1[0:1, :][None]
                     + cphi[..., None] * w1[1:2, :][None]
                     + sphi[..., None] * w1[2:3, :][None]
                     + b1[None])
    hf = h3.reshape(SB * NVP, MSG)
    f1f = f1.reshape(SB * NVP, MSG)
    f = (jnp.dot(f1f, w2_ref[...], preferred_element_type=jnp.float32)
         + b2_ref[...])
    g = hf * f
    m = jnp.dot(g, wo_ref[...], preferred_element_type=jnp.float32)
    m3 = m.reshape(SB, NVP, 3 * MSG)
    bo = bo_ref[...]
    gg0 = g_ref[0, :, :, 0:1]
    gg1 = g_ref[0, :, :, 1:2]
    gg2 = g_ref[0, :, :, 2:3]
    contrib = ((m3[..., 0:MSG] + bo[0:1, 0:MSG][None]) * gg0
               + (m3[..., MSG:2 * MSG] + bo[0:1, MSG:2 * MSG][None]) * gg1
               + (m3[..., 2 * MSG:] + bo[0:1, 2 * MSG:][None]) * gg2)
    part = jnp.sum(contrib, axis=0)[None]

    @pl.when(sblk == 0)
    def _():
        o_ref[...] = part

    @pl.when(sblk > 0)
    def _():
        o_ref[...] += part


def _post_body(agg_ref, res_ref, x_ref, w1_ref, b1_ref, w2_ref, b2_ref,
               w3_ref, b3_ref, o_ref):
    b, n = x_ref.shape[0], x_ref.shape[1]
    x = x_ref[...].reshape(b * n, 4)
    aug = (agg_ref[...].reshape(b * n, MSG) * (1.0 / (NV - 1))
           + res_ref[...].reshape(b * n, MSG))
    h1 = jax.nn.relu(
        jnp.dot(aug, w1_ref[...], preferred_element_type=jnp.float32)
        + b1_ref[...])
    h2 = jax.nn.relu(
        jnp.dot(h1, w2_ref[...], preferred_element_type=jnp.float32)
        + b2_ref[...])
    pred = (jnp.dot(h2, w3_ref[...], preferred_element_type=jnp.float32)
            + b3_ref[...])
    vx, vy = x[:, 2:3], x[:, 3:4]
    r = jnp.sqrt(vx * vx + vy * vy)
    safe = r > 0.0
    rinvsafe = 1.0 / jnp.where(safe, r, 1.0)
    c = jnp.where(safe, vx * rinvsafe, 1.0)
    s = jnp.where(safe, vy * rinvsafe, 0.0)
    p0, p1, p2, p3 = pred[:, 0:1], pred[:, 1:2], pred[:, 2:3], pred[:, 3:4]
    pg = jnp.concatenate(
        [c * p0 - s * p1, s * p0 + c * p1,
         c * p2 - s * p3, s * p2 + c * p3], axis=1)
    o_ref[...] = (x + pg).reshape(b, n, 4)


def kernel(inputs, hidden, edges, predicted_field, charge_emb,
           res1_W, res1_b, ef_attr_W, ef_attr_b,
           ef_pos_W1, ef_pos_b1, ef_pos_W2, ef_pos_b2,
           ef_out_W, ef_out_b, out_W1, out_b1, out_W2, out_b2,
           out_W3, out_b3):
    del hidden
    B = inputs.shape[0]
    f32 = jnp.float32

    pad_n = NVP - NV
    x_p = jnp.pad(inputs, ((0, 0), (0, pad_n), (0, 0)))
    pf_p = jnp.pad(predicted_field, ((0, 0), (0, pad_n), (0, 0)))
    ce_p = jnp.pad(charge_emb, ((0, 0), (0, pad_n), (0, 0)))
    pos_p = x_p[:, :, 0:2]
    pos_pt = jnp.transpose(pos_p, (0, 2, 1))

    Wa = ef_attr_W
    w_send = jnp.concatenate([Wa[18:24], Wa[9:18], Wa[40:56]], axis=0)
    w_recv = jnp.concatenate([-Wa[18:24], Wa[0:9], Wa[24:40]], axis=0)

    wo = ef_out_W.reshape(MSG, MSG, 3).transpose(0, 2, 1).reshape(MSG, 3 * MSG)
    bo = ef_out_b.reshape(MSG, 3).T.reshape(1, 3 * MSG)

    e3 = edges[:, :, 1:4].astype(f32)
    gtmp = e3.reshape(B, NV - 1, NV, 3)
    gtmp = jnp.pad(gtmp, ((0, 0), (0, 0), (1, 0), (0, 0)))
    gtmp = gtmp.reshape(B, (NV - 1) * (NV + 1), 3)
    gtmp = jnp.pad(gtmp, ((0, 0), (0, 1), (0, 0)))
    gates = gtmp.reshape(B, NV, NV, 3)
    gates = jnp.pad(gates, ((0, 0), (0, pad_n), (0, pad_n), (0, 0)))

    b2d = lambda v: v.reshape(1, -1).astype(f32)

    grid_b = (B // BB,)
    node_spec = lambda last: pl.BlockSpec((BB, NVP, last), lambda i: (i, 0, 0))
    full = lambda a: pl.BlockSpec(a.shape, lambda i: (0,) * a.ndim)
    ws = w_send
    wr = w_recv
    wres = res1_W
    a_s, a_r, res_x = pl.pallas_call(
        _prep_body,
        grid=grid_b,
        in_specs=[node_spec(4), node_spec(2), node_spec(16),
                  full(ws), full(wr), full(wres),
                  pl.BlockSpec((1, MSG), lambda i: (0, 0)),
                  pl.BlockSpec((1, MSG), lambda i: (0, 0))],
        out_specs=[node_spec(MSG), node_spec(MSG), node_spec(MSG)],
        out_shape=[jax.ShapeDtypeStruct((B, NVP, MSG), f32)] * 3,
    )(x_p, pf_p, ce_p, ws, wr, wres, b2d(ef_attr_b), b2d(res1_b))

    agg = pl.pallas_call(
        _edge_body,
        grid=(B, NSB),
        in_specs=[
            pl.BlockSpec((1, SB, MSG), lambda b, s: (b, s, 0)),
            pl.BlockSpec((1, NVP, MSG), lambda b, s: (b, 0, 0)),
            pl.BlockSpec((1, SB, 2), lambda b, s: (b, s, 0)),
            pl.BlockSpec((1, 2, NVP), lambda b, s: (b, 0, 0)),
            pl.BlockSpec((1, SB, NVP, 3), lambda b, s: (b, s, 0, 0)),
            pl.BlockSpec((3, MSG), lambda b, s: (0, 0)),
            pl.BlockSpec((1, MSG), lambda b, s: (0, 0)),
            pl.BlockSpec((MSG, MSG), lambda b, s: (0, 0)),
            pl.BlockSpec((1, MSG), lambda b, s: (0, 0)),
            pl.BlockSpec((MSG, 3 * MSG), lambda b, s: (0, 0)),
            pl.BlockSpec((1, 3 * MSG), lambda b, s: (0, 0)),
        ],
        out_specs=pl.BlockSpec((1, NVP, MSG), lambda b, s: (b, 0, 0)),
        out_shape=jax.ShapeDtypeStruct((B, NVP, MSG), f32),
        compiler_params=pltpu.CompilerParams(
            dimension_semantics=("parallel", "arbitrary")),
    )(a_s, a_r, pos_p, pos_pt, gates,
      ef_pos_W1, b2d(ef_pos_b1), ef_pos_W2, b2d(ef_pos_b2), wo, bo)

    out_p = pl.pallas_call(
        _post_body,
        grid=grid_b,
        in_specs=[node_spec(MSG), node_spec(MSG), node_spec(4),
                  full(out_W1), pl.BlockSpec((1, MSG), lambda i: (0, 0)),
                  full(out_W2), pl.BlockSpec((1, MSG), lambda i: (0, 0)),
                  full(out_W3), pl.BlockSpec((1, 4), lambda i: (0, 0))],
        out_specs=node_spec(4),
        out_shape=jax.ShapeDtypeStruct((B, NVP, 4), f32),
    )(agg, res_x, x_p,
      out_W1, b2d(out_b1), out_W2, b2d(out_b2), out_W3, b2d(out_b3))

    return out_p[:, :NV, :]


# speedup vs baseline: 2.2141x; 2.2141x over previous
import numpy as np
import jax
import jax.numpy as jnp
from jax.experimental import pallas as pl
from jax.experimental.pallas import tpu as pltpu

NV = 100
NVP = 104
ND = 2
MSG = 64
SB = 8
NSB = NVP // SB


def _cs_from_vel(vx, vy):
    r = jnp.sqrt(vx * vx + vy * vy)
    safe = r > 0.0
    rinv = 1.0 / jnp.where(safe, r, 1.0)
    c = jnp.where(safe, vx * rinv, 1.0)
    s = jnp.where(safe, vy * rinv, 0.0)
    return c, s


def _prep_body(x_ref, pf_ref, ce_ref, e_ref, ws_ref, wr_ref, wres_ref,
               battr_ref, bres_ref,
               as_ref, ar_ref, res_ref, ps_ref, pt_ref, g_ref):
    x = x_ref[0]
    pf = pf_ref[0]
    ce = ce_ref[0]
    px, py = x[:, 0:1], x[:, 1:2]
    vx, vy = x[:, 2:3], x[:, 3:4]
    fx, fy = pf[:, 0:1], pf[:, 1:2]
    c, s = _cs_from_vel(vx, vy)
    lp0 = c * px + s * py
    lp1 = -s * px + c * py
    lv0 = c * vx + s * vy
    lv1 = -s * vx + c * vy
    lf0 = c * fx + s * fy
    lf1 = -s * fx + c * fy
    nvel = jnp.sqrt(vx * vx + vy * vy + 1e-8)
    npos = jnp.sqrt(px * px + py * py + 1e-8)
    nfld = jnp.sqrt(fx * fx + fy * fy + 1e-8)
    f31 = jnp.concatenate(
        [lp0, lp1, lv0, lv1, lf0, lf1,
         nvel, npos, c, s, lp0, lp1, lf0, lf1, nfld, ce], axis=1)
    a_s = jnp.dot(f31, ws_ref[...], preferred_element_type=jnp.float32)
    a_r = (jnp.dot(f31, wr_ref[...], preferred_element_type=jnp.float32)
           + battr_ref[...])
    rx = (jnp.dot(f31, wres_ref[...], preferred_element_type=jnp.float32)
          + bres_ref[...])
    padn = ((0, NVP - NV), (0, 0))
    as_ref[...] = jnp.pad(a_s, padn)[None]
    ar_ref[...] = jnp.pad(a_r, padn)[None]
    res_ref[...] = jnp.pad(rx, padn)[None]
    pos = jnp.pad(jnp.concatenate([px, py], axis=1), padn)
    ps_ref[...] = pos[None]
    pt_ref[...] = pos.T[None]
    e = e_ref[0]
    ii = jax.lax.broadcasted_iota(jnp.int32, (NV, NVP), 0)
    jj = jax.lax.broadcasted_iota(jnp.int32, (NV, NVP), 1)
    zcol = jnp.zeros((NV, 1), jnp.float32)
    for ch in range(3):
        gv = e[:, :, ch + 1]
        ap = jnp.pad(gv, ((0, 0), (0, NVP - NV + 1)))
        rolled = jnp.concatenate([zcol, ap[:, :NVP - 1]], axis=1)
        g = jnp.where(jj < ii, ap, jnp.where(jj > ii, rolled, 0.0))
        g_ref[0, ch] = jnp.pad(g, ((0, NVP - NV), (0, 0)))


def _edge_body(as_ref, ar_ref, ps_ref, prt_ref, g_ref,
               w1_ref, b1_ref, w2_ref, b2_ref, wo_ref, bo_ref, o_ref):
    sblk = pl.program_id(1)
    a_s = as_ref[0]
    a_r = ar_ref[0]
    h3 = jax.nn.relu(a_s[:, None, :] + a_r[None, :, :])
    psx, psy = ps_ref[0, :, 0:1], ps_ref[0, :, 1:2]
    prx, pry = prt_ref[0, 0:1, :], prt_ref[0, 1:2, :]
    dx = psx - prx
    dy = psy - pry
    d2 = dx * dx + dy * dy
    dist = jnp.sqrt(d2 + 1e-8)
    rr = jnp.sqrt(d2)
    safe = rr > 0.0
    inv = jnp.where(safe, 1.0 / jnp.where(safe, rr, 1.0), 0.0)
    cphi = jnp.where(safe, dx * inv, 1.0)
    sphi = dy * inv
    w1 = w1_ref[...]
    b1 = b1_ref[...]
    f1 = jax.nn.relu(dist[..., None] * w1[0:1, :][None]
                     + cphi[..., None] * w1[1:2, :][None]
                     + sphi[..., None] * w1[2:3, :][None]
                     + b1[None])
    hf = h3.reshape(SB * NVP, MSG)
    f1f = f1.reshape(SB * NVP, MSG)
    f = (jnp.dot(f1f, w2_ref[...], preferred_element_type=jnp.float32)
         + b2_ref[...])
    g = hf * f
    m = jnp.dot(g, wo_ref[...], preferred_element_type=jnp.float32)
    m3 = m.reshape(SB, NVP, 3 * MSG)
    bo = bo_ref[...]
    gg0 = g_ref[0, 0][..., None]
    gg1 = g_ref[0, 1][..., None]
    gg2 = g_ref[0, 2][..., None]
    contrib = ((m3[..., 0:MSG] + bo[0:1, 0:MSG][None]) * gg0
               + (m3[..., MSG:2 * MSG] + bo[0:1, MSG:2 * MSG][None]) * gg1
               + (m3[..., 2 * MSG:] + bo[0:1, 2 * MSG:][None]) * gg2)
    part = jnp.sum(contrib, axis=0)[None]

    @pl.when(sblk == 0)
    def _():
        o_ref[...] = part

    @pl.when(sblk > 0)
    def _():
        o_ref[...] += part


def _post_body(agg_ref, res_ref, x_ref, w1_ref, b1_ref, w2_ref, b2_ref,
               w3_ref, b3_ref, o_ref):
    x = x_ref[0]
    aug = (agg_ref[0, :NV, :] * (1.0 / (NV - 1)) + res_ref[0, :NV, :])
    h1 = jax.nn.relu(
        jnp.dot(aug, w1_ref[...], preferred_element_type=jnp.float32)
        + b1_ref[...])
    h2 = jax.nn.relu(
        jnp.dot(h1, w2_ref[...], preferred_element_type=jnp.float32)
        + b2_ref[...])
    pred = (jnp.dot(h2, w3_ref[...], preferred_element_type=jnp.float32)
            + b3_ref[...])
    c, s = _cs_from_vel(x[:, 2:3], x[:, 3:4])
    p0, p1, p2, p3 = pred[:, 0:1], pred[:, 1:2], pred[:, 2:3], pred[:, 3:4]
    pg = jnp.concatenate(
        [c * p0 - s * p1, s * p0 + c * p1,
         c * p2 - s * p3, s * p2 + c * p3], axis=1)
    o_ref[...] = (x + pg)[None]


def kernel(inputs, hidden, edges, predicted_field, charge_emb,
           res1_W, res1_b, ef_attr_W, ef_attr_b,
           ef_pos_W1, ef_pos_b1, ef_pos_W2, ef_pos_b2,
           ef_out_W, ef_out_b, out_W1, out_b1, out_W2, out_b2,
           out_W3, out_b3):
    del hidden
    B = inputs.shape[0]
    f32 = jnp.float32

    Wa = ef_attr_W
    w_send = jnp.concatenate([Wa[18:24], Wa[9:18], Wa[40:56]], axis=0)
    w_recv = jnp.concatenate([-Wa[18:24], Wa[0:9], Wa[24:40]], axis=0)

    wo = ef_out_W.reshape(MSG, MSG, 3).transpose(0, 2, 1).reshape(MSG, 3 * MSG)
    bo = ef_out_b.reshape(MSG, 3).T.reshape(1, 3 * MSG)

    e4 = edges.reshape(B, NV, NV - 1, 4)
    b2d = lambda v: v.reshape(1, -1).astype(f32)

    full = lambda a: pl.BlockSpec(a.shape, lambda i: (0,) * a.ndim)

    a_s, a_r, res_x, pos_p, pos_pt, gates = pl.pallas_call(
        _prep_body,
        grid=(B,),
        in_specs=[pl.BlockSpec((1, NV, 4), lambda i: (i, 0, 0)),
                  pl.BlockSpec((1, NV, 2), lambda i: (i, 0, 0)),
                  pl.BlockSpec((1, NV, 16), lambda i: (i, 0, 0)),
                  pl.BlockSpec((1, NV, NV - 1, 4), lambda i: (i, 0, 0, 0)),
                  full(w_send), full(w_recv), full(res1_W),
                  pl.BlockSpec((1, MSG), lambda i: (0, 0)),
                  pl.BlockSpec((1, MSG), lambda i: (0, 0))],
        out_specs=[pl.BlockSpec((1, NVP, MSG), lambda i: (i, 0, 0)),
                   pl.BlockSpec((1, NVP, MSG), lambda i: (i, 0, 0)),
                   pl.BlockSpec((1, NVP, MSG), lambda i: (i, 0, 0)),
                   pl.BlockSpec((1, NVP, 2), lambda i: (i, 0, 0)),
                   pl.BlockSpec((1, 2, NVP), lambda i: (i, 0, 0)),
                   pl.BlockSpec((1, 3, NVP, NVP), lambda i: (i, 0, 0, 0))],
        out_shape=[jax.ShapeDtypeStruct((B, NVP, MSG), f32),
                   jax.ShapeDtypeStruct((B, NVP, MSG), f32),
                   jax.ShapeDtypeStruct((B, NVP, MSG), f32),
                   jax.ShapeDtypeStruct((B, NVP, 2), f32),
                   jax.ShapeDtypeStruct((B, 2, NVP), f32),
                   jax.ShapeDtypeStruct((B, 3, NVP, NVP), f32)],
        compiler_params=pltpu.CompilerParams(
            dimension_semantics=("parallel",)),
    )(inputs, predicted_field, charge_emb, e4,
      w_send, w_recv, res1_W, b2d(ef_attr_b), b2d(res1_b))

    agg = pl.pallas_call(
        _edge_body,
        grid=(B, NSB),
        in_specs=[
            pl.BlockSpec((1, SB, MSG), lambda b, s: (b, s, 0)),
            pl.BlockSpec((1, NVP, MSG), lambda b, s: (b, 0, 0)),
            pl.BlockSpec((1, SB, 2), lambda b, s: (b, s, 0)),
            pl.BlockSpec((1, 2, NVP), lambda b, s: (b, 0, 0)),
            pl.BlockSpec((1, 3, SB, NVP), lambda b, s: (b, 0, s, 0)),
            pl.BlockSpec((3, MSG), lambda b, s: (0, 0)),
            pl.BlockSpec((1, MSG), lambda b, s: (0, 0)),
            pl.BlockSpec((MSG, MSG), lambda b, s: (0, 0)),
            pl.BlockSpec((1, MSG), lambda b, s: (0, 0)),
            pl.BlockSpec((MSG, 3 * MSG), lambda b, s: (0, 0)),
            pl.BlockSpec((1, 3 * MSG), lambda b, s: (0, 0)),
        ],
        out_specs=pl.BlockSpec((1, NVP, MSG), lambda b, s: (b, 0, 0)),
        out_shape=jax.ShapeDtypeStruct((B, NVP, MSG), f32),
        compiler_params=pltpu.CompilerParams(
            dimension_semantics=("parallel", "arbitrary")),
    )(a_s, a_r, pos_p, pos_pt, gates,
      ef_pos_W1, b2d(ef_pos_b1), ef_pos_W2, b2d(ef_pos_b2), wo, bo)

    out = pl.pallas_call(
        _post_body,
        grid=(B,),
        in_specs=[pl.BlockSpec((1, NVP, MSG), lambda i: (i, 0, 0)),
                  pl.BlockSpec((1, NVP, MSG), lambda i: (i, 0, 0)),
                  pl.BlockSpec((1, NV, 4), lambda i: (i, 0, 0)),
                  full(out_W1), pl.BlockSpec((1, MSG), lambda i: (0, 0)),
                  full(out_W2), pl.BlockSpec((1, MSG), lambda i: (0, 0)),
                  full(out_W3), pl.BlockSpec((1, 4), lambda i: (0, 0))],
        out_specs=pl.BlockSpec((1, NV, 4), lambda i: (i, 0, 0)),
        out_shape=jax.ShapeDtypeStruct((B, NV, 4), f32),
        compiler_params=pltpu.CompilerParams(
            dimension_semantics=("parallel",)),
    )(agg, res_x, inputs,
      out_W1, b2d(out_b1), out_W2, b2d(out_b2), out_W3, b2d(out_b3))

    return out
